# baseline (speedup 1.0000x reference)
import jax
import jax.numpy as jnp
from jax import lax
from jax.experimental import pallas as pl
from jax.experimental.pallas import tpu as pltpu

N_NODES = 10000
D_IN = 128
D_H = 64
HEADS = 8
N_OUT = 8
ROW_BLK = 1000


def _prologue1_body(v_ref, fw_ref, fb_ref, Up_ref, cv_ref, Ws_ref,
                    x_ref, pq_ref, self_ref):
    h = v_ref[...] @ fw_ref[...] + fb_ref[...]
    h = jnp.where(h > 0, h, jnp.exp(h) - 1.0)
    x_ref[...] = h
    pq_ref[...] = h @ Up_ref[...] - cv_ref[...]
    self_ref[...] = h @ Ws_ref[...]


def _prologue1(verts, fc0_w, fc0_b, Upad, cvec, Wself):
    grid = (N_NODES // ROW_BLK,)
    return pl.pallas_call(
        _prologue1_body,
        grid=grid,
        in_specs=[
            pl.BlockSpec((ROW_BLK, D_IN), lambda i: (i, 0)),
            pl.BlockSpec((D_IN, D_H), lambda i: (0, 0)),
            pl.BlockSpec((D_H,), lambda i: (0,)),
            pl.BlockSpec((D_H, 16), lambda i: (0, 0)),
            pl.BlockSpec((16,), lambda i: (0,)),
            pl.BlockSpec((D_H, D_H), lambda i: (0, 0)),
        ],
        out_specs=[
            pl.BlockSpec((ROW_BLK, D_H), lambda i: (i, 0)),
            pl.BlockSpec((ROW_BLK, 16), lambda i: (i, 0)),
            pl.BlockSpec((ROW_BLK, D_H), lambda i: (i, 0)),
        ],
        out_shape=[
            jax.ShapeDtypeStruct((N_NODES, D_H), jnp.float32),
            jax.ShapeDtypeStruct((N_NODES, 16), jnp.float32),
            jax.ShapeDtypeStruct((N_NODES, D_H), jnp.float32),
        ],
    )(verts, fc0_w, fc0_b, Upad, cvec, Wself)


def _mid_body(ag_ref, dg_ref, self_ref, b_ref, Up_ref, cv_ref, Ws_ref,
              x_ref, pq_ref, self2_ref):
    x = (ag_ref[...] + self_ref[...]) / (dg_ref[...] + 1.0) + b_ref[...]
    x = jnp.where(x > 0, x, jnp.exp(x) - 1.0)
    x_ref[...] = x
    pq_ref[...] = x @ Up_ref[...] - cv_ref[...]
    self2_ref[...] = x @ Ws_ref[...]


def _mid(agg, deg3, self_msg, b, Upad, cvec, Wself):
    grid = (N_NODES // ROW_BLK,)
    return pl.pallas_call(
        _mid_body,
        grid=grid,
        in_specs=[
            pl.BlockSpec((ROW_BLK, D_H), lambda i: (i, 0)),
            pl.BlockSpec((ROW_BLK, 1), lambda i: (i, 0)),
            pl.BlockSpec((ROW_BLK, D_H), lambda i: (i, 0)),
            pl.BlockSpec((D_H,), lambda i: (0,)),
            pl.BlockSpec((D_H, 16), lambda i: (0, 0)),
            pl.BlockSpec((16,), lambda i: (0,)),
            pl.BlockSpec((D_H, D_H), lambda i: (0, 0)),
        ],
        out_specs=[
            pl.BlockSpec((ROW_BLK, D_H), lambda i: (i, 0)),
            pl.BlockSpec((ROW_BLK, 16), lambda i: (i, 0)),
            pl.BlockSpec((ROW_BLK, D_H), lambda i: (i, 0)),
        ],
        out_shape=[
            jax.ShapeDtypeStruct((N_NODES, D_H), jnp.float32),
            jax.ShapeDtypeStruct((N_NODES, 16), jnp.float32),
            jax.ShapeDtypeStruct((N_NODES, D_H), jnp.float32),
        ],
    )(agg, deg3, self_msg, b, Upad, cvec, Wself)


def _final_body(ag_ref, dg_ref, self_ref, b_ref, lw_ref, lb_ref, out_ref):
    x = (ag_ref[...] + self_ref[...]) / (dg_ref[...] + 1.0) + b_ref[...]
    x = jnp.where(x > 0, x, jnp.exp(x) - 1.0)
    out_ref[...] = jnp.tanh(x @ lw_ref[...] + lb_ref[...])


def _final(agg, deg3, self_msg, b, lin_w, lin_b):
    grid = (N_NODES // ROW_BLK,)
    return pl.pallas_call(
        _final_body,
        grid=grid,
        in_specs=[
            pl.BlockSpec((ROW_BLK, D_H), lambda i: (i, 0)),
            pl.BlockSpec((ROW_BLK, 1), lambda i: (i, 0)),
            pl.BlockSpec((ROW_BLK, D_H), lambda i: (i, 0)),
            pl.BlockSpec((D_H,), lambda i: (0,)),
            pl.BlockSpec((D_H, N_OUT), lambda i: (0, 0)),
            pl.BlockSpec((N_OUT,), lambda i: (0,)),
        ],
        out_specs=pl.BlockSpec((ROW_BLK, N_OUT), lambda i: (i, 0)),
        out_shape=jax.ShapeDtypeStruct((N_NODES, N_OUT), jnp.float32),
    )(agg, deg3, self_msg, b, lin_w, lin_b)


def _edge_phase(x, pq, W, edges):
    src, dst = edges[0], edges[1]
    mask = (src != dst).astype(jnp.float32)
    q = jax.nn.softmax(pq[src, :HEADS] - pq[dst, HEADS:], axis=1)
    q = q * mask[:, None]
    m = (x[src] @ W).reshape(-1, HEADS, D_H)
    msg = (m * q[:, :, None]).sum(axis=1)
    agg = jax.ops.segment_sum(msg, dst, num_segments=N_NODES)
    deg = jax.ops.segment_sum(mask, dst, num_segments=N_NODES)
    return agg, deg[:, None]


def _prep_layer(W, U, c):
    Upad = jnp.concatenate([U, U], axis=1)
    cvec = jnp.concatenate([jnp.zeros((HEADS,), jnp.float32), c])
    qs = jax.nn.softmax(c)
    Wself = (W.reshape(D_H, HEADS, D_H) * qs[None, :, None]).sum(axis=1)
    return Upad, cvec, Wself


def kernel(verts, edges, fc0_w, fc0_b, conv1_W, conv1_U, conv1_c, conv1_b,
           conv2_W, conv2_U, conv2_c, conv2_b, lin_w, lin_b):
    Upad1, cvec1, Wself1 = _prep_layer(conv1_W, conv1_U, conv1_c)
    Upad2, cvec2, Wself2 = _prep_layer(conv2_W, conv2_U, conv2_c)

    x0, pq1, self1 = _prologue1(verts, fc0_w, fc0_b, Upad1, cvec1, Wself1)
    agg1, deg1 = _edge_phase(x0, pq1, conv1_W, edges)
    x1, pq2, self2 = _mid(agg1, deg1, self1, conv1_b, Upad2, cvec2, Wself2)
    agg2, deg2 = _edge_phase(x1, pq2, conv2_W, edges)
    return _final(agg2, deg2, self2, conv2_b, lin_w, lin_b)


# speedup vs baseline: 52.3248x; 52.3248x over previous
import jax
import jax.numpy as jnp
from jax import lax
from jax.experimental import pallas as pl
from jax.experimental.pallas import tpu as pltpu

N_NODES = 10000
D_IN = 128
D_H = 64
HEADS = 8
N_OUT = 8
ROW_BLK = 1000


def _prologue1_body(v_ref, fw_ref, fb_ref, Up_ref, cv_ref, Ws_ref,
                    x_ref, pq_ref, self_ref):
    h = v_ref[...] @ fw_ref[...] + fb_ref[...]
    h = jnp.where(h > 0, h, jnp.exp(h) - 1.0)
    x_ref[...] = h
    pq_ref[...] = h @ Up_ref[...] - cv_ref[...]
    self_ref[...] = h @ Ws_ref[...]


def _prologue1(verts, fc0_w, fc0_b, Upad, cvec, Wself):
    grid = (N_NODES // ROW_BLK,)
    return pl.pallas_call(
        _prologue1_body,
        grid=grid,
        in_specs=[
            pl.BlockSpec((ROW_BLK, D_IN), lambda i: (i, 0)),
            pl.BlockSpec((D_IN, D_H), lambda i: (0, 0)),
            pl.BlockSpec((D_H,), lambda i: (0,)),
            pl.BlockSpec((D_H, 16), lambda i: (0, 0)),
            pl.BlockSpec((16,), lambda i: (0,)),
            pl.BlockSpec((D_H, D_H), lambda i: (0, 0)),
        ],
        out_specs=[
            pl.BlockSpec((ROW_BLK, D_H), lambda i: (i, 0)),
            pl.BlockSpec((ROW_BLK, 16), lambda i: (i, 0)),
            pl.BlockSpec((ROW_BLK, D_H), lambda i: (i, 0)),
        ],
        out_shape=[
            jax.ShapeDtypeStruct((N_NODES, D_H), jnp.float32),
            jax.ShapeDtypeStruct((N_NODES, 16), jnp.float32),
            jax.ShapeDtypeStruct((N_NODES, D_H), jnp.float32),
        ],
    )(verts, fc0_w, fc0_b, Upad, cvec, Wself)


def _mid_body(ag_ref, dg_ref, self_ref, b_ref, Up_ref, cv_ref, Ws_ref,
              x_ref, pq_ref, self2_ref):
    x = (ag_ref[...] + self_ref[...]) / (dg_ref[...] + 1.0) + b_ref[...]
    x = jnp.where(x > 0, x, jnp.exp(x) - 1.0)
    x_ref[...] = x
    pq_ref[...] = x @ Up_ref[...] - cv_ref[...]
    self2_ref[...] = x @ Ws_ref[...]


def _mid(agg, deg3, self_msg, b, Upad, cvec, Wself):
    grid = (N_NODES // ROW_BLK,)
    return pl.pallas_call(
        _mid_body,
        grid=grid,
        in_specs=[
            pl.BlockSpec((ROW_BLK, D_H), lambda i: (i, 0)),
            pl.BlockSpec((ROW_BLK, 1), lambda i: (i, 0)),
            pl.BlockSpec((ROW_BLK, D_H), lambda i: (i, 0)),
            pl.BlockSpec((D_H,), lambda i: (0,)),
            pl.BlockSpec((D_H, 16), lambda i: (0, 0)),
            pl.BlockSpec((16,), lambda i: (0,)),
            pl.BlockSpec((D_H, D_H), lambda i: (0, 0)),
        ],
        out_specs=[
            pl.BlockSpec((ROW_BLK, D_H), lambda i: (i, 0)),
            pl.BlockSpec((ROW_BLK, 16), lambda i: (i, 0)),
            pl.BlockSpec((ROW_BLK, D_H), lambda i: (i, 0)),
        ],
        out_shape=[
            jax.ShapeDtypeStruct((N_NODES, D_H), jnp.float32),
            jax.ShapeDtypeStruct((N_NODES, 16), jnp.float32),
            jax.ShapeDtypeStruct((N_NODES, D_H), jnp.float32),
        ],
    )(agg, deg3, self_msg, b, Upad, cvec, Wself)


def _final_body(ag_ref, dg_ref, self_ref, b_ref, lw_ref, lb_ref, out_ref):
    x = (ag_ref[...] + self_ref[...]) / (dg_ref[...] + 1.0) + b_ref[...]
    x = jnp.where(x > 0, x, jnp.exp(x) - 1.0)
    out_ref[...] = jnp.tanh(x @ lw_ref[...] + lb_ref[...])


def _final(agg, deg3, self_msg, b, lin_w, lin_b):
    grid = (N_NODES // ROW_BLK,)
    return pl.pallas_call(
        _final_body,
        grid=grid,
        in_specs=[
            pl.BlockSpec((ROW_BLK, D_H), lambda i: (i, 0)),
            pl.BlockSpec((ROW_BLK, 1), lambda i: (i, 0)),
            pl.BlockSpec((ROW_BLK, D_H), lambda i: (i, 0)),
            pl.BlockSpec((D_H,), lambda i: (0,)),
            pl.BlockSpec((D_H, N_OUT), lambda i: (0, 0)),
            pl.BlockSpec((N_OUT,), lambda i: (0,)),
        ],
        out_specs=pl.BlockSpec((ROW_BLK, N_OUT), lambda i: (i, 0)),
        out_shape=jax.ShapeDtypeStruct((N_NODES, N_OUT), jnp.float32),
    )(agg, deg3, self_msg, b, lin_w, lin_b)


def _edge_phase(x, U, c, W, edges):
    src, dst = edges[0], edges[1]
    mask = (src != dst).astype(jnp.float32)
    xj = x[src]
    xi = x[dst]
    q = jax.nn.softmax((xj - xi) @ U + c, axis=1)
    q = q * mask[:, None]
    m = (xj @ W).reshape(-1, HEADS, D_H)
    msg = (m * q[:, :, None]).sum(axis=1)
    agg = jax.ops.segment_sum(msg, dst, num_segments=N_NODES)
    deg = jax.ops.segment_sum(mask, dst, num_segments=N_NODES)
    return agg, deg[:, None]


def _prep_layer(W, U, c):
    Upad = jnp.concatenate([U, U], axis=1)
    cvec = jnp.concatenate([jnp.zeros((HEADS,), jnp.float32), c])
    qs = jax.nn.softmax(c)
    Wself = (W.reshape(D_H, HEADS, D_H) * qs[None, :, None]).sum(axis=1)
    return Upad, cvec, Wself


def kernel(verts, edges, fc0_w, fc0_b, conv1_W, conv1_U, conv1_c, conv1_b,
           conv2_W, conv2_U, conv2_c, conv2_b, lin_w, lin_b):
    Upad1, cvec1, Wself1 = _prep_layer(conv1_W, conv1_U, conv1_c)
    Upad2, cvec2, Wself2 = _prep_layer(conv2_W, conv2_U, conv2_c)

    x0, pq1, self1 = _prologue1(verts, fc0_w, fc0_b, Upad1, cvec1, Wself1)
    agg1, deg1 = _edge_phase(x0, conv1_U, conv1_c, conv1_W, edges)
    x1, pq2, self2 = _mid(agg1, deg1, self1, conv1_b, Upad2, cvec2, Wself2)
    agg2, deg2 = _edge_phase(x1, conv2_U, conv2_c, conv2_W, edges)
    return _final(agg2, deg2, self2, conv2_b, lin_w, lin_b)


# speedup vs baseline: 53.5301x; 1.0230x over previous
import jax
import jax.numpy as jnp
from jax import lax
from jax.experimental import pallas as pl
from jax.experimental.pallas import tpu as pltpu

N_NODES = 10000
D_IN = 128
D_H = 64
HEADS = 8
N_OUT = 8
ROW_BLK = 1000


def _prologue1_body(v_ref, fw_ref, fb_ref, Up_ref, cv_ref, Ws_ref,
                    x_ref, pq_ref, self_ref):
    h = v_ref[...] @ fw_ref[...] + fb_ref[...]
    h = jnp.where(h > 0, h, jnp.exp(h) - 1.0)
    x_ref[...] = h
    pq_ref[...] = h @ Up_ref[...] - cv_ref[...]
    self_ref[...] = h @ Ws_ref[...]


def _prologue1(verts, fc0_w, fc0_b, Upad, cvec, Wself):
    grid = (N_NODES // ROW_BLK,)
    return pl.pallas_call(
        _prologue1_body,
        grid=grid,
        in_specs=[
            pl.BlockSpec((ROW_BLK, D_IN), lambda i: (i, 0)),
            pl.BlockSpec((D_IN, D_H), lambda i: (0, 0)),
            pl.BlockSpec((D_H,), lambda i: (0,)),
            pl.BlockSpec((D_H, 16), lambda i: (0, 0)),
            pl.BlockSpec((16,), lambda i: (0,)),
            pl.BlockSpec((D_H, D_H), lambda i: (0, 0)),
        ],
        out_specs=[
            pl.BlockSpec((ROW_BLK, D_H), lambda i: (i, 0)),
            pl.BlockSpec((ROW_BLK, 16), lambda i: (i, 0)),
            pl.BlockSpec((ROW_BLK, D_H), lambda i: (i, 0)),
        ],
        out_shape=[
            jax.ShapeDtypeStruct((N_NODES, D_H), jnp.float32),
            jax.ShapeDtypeStruct((N_NODES, 16), jnp.float32),
            jax.ShapeDtypeStruct((N_NODES, D_H), jnp.float32),
        ],
    )(verts, fc0_w, fc0_b, Upad, cvec, Wself)


def _mid_body(ag_ref, dg_ref, self_ref, b_ref, Up_ref, cv_ref, Ws_ref,
              x_ref, pq_ref, self2_ref):
    x = (ag_ref[...] + self_ref[...]) / (dg_ref[...] + 1.0) + b_ref[...]
    x = jnp.where(x > 0, x, jnp.exp(x) - 1.0)
    x_ref[...] = x
    pq_ref[...] = x @ Up_ref[...] - cv_ref[...]
    self2_ref[...] = x @ Ws_ref[...]


def _mid(agg, deg3, self_msg, b, Upad, cvec, Wself):
    grid = (N_NODES // ROW_BLK,)
    return pl.pallas_call(
        _mid_body,
        grid=grid,
        in_specs=[
            pl.BlockSpec((ROW_BLK, D_H), lambda i: (i, 0)),
            pl.BlockSpec((ROW_BLK, 1), lambda i: (i, 0)),
            pl.BlockSpec((ROW_BLK, D_H), lambda i: (i, 0)),
            pl.BlockSpec((D_H,), lambda i: (0,)),
            pl.BlockSpec((D_H, 16), lambda i: (0, 0)),
            pl.BlockSpec((16,), lambda i: (0,)),
            pl.BlockSpec((D_H, D_H), lambda i: (0, 0)),
        ],
        out_specs=[
            pl.BlockSpec((ROW_BLK, D_H), lambda i: (i, 0)),
            pl.BlockSpec((ROW_BLK, 16), lambda i: (i, 0)),
            pl.BlockSpec((ROW_BLK, D_H), lambda i: (i, 0)),
        ],
        out_shape=[
            jax.ShapeDtypeStruct((N_NODES, D_H), jnp.float32),
            jax.ShapeDtypeStruct((N_NODES, 16), jnp.float32),
            jax.ShapeDtypeStruct((N_NODES, D_H), jnp.float32),
        ],
    )(agg, deg3, self_msg, b, Upad, cvec, Wself)


def _final_body(ag_ref, dg_ref, self_ref, b_ref, lw_ref, lb_ref, out_ref):
    x = (ag_ref[...] + self_ref[...]) / (dg_ref[...] + 1.0) + b_ref[...]
    x = jnp.where(x > 0, x, jnp.exp(x) - 1.0)
    out_ref[...] = jnp.tanh(x @ lw_ref[...] + lb_ref[...])


def _final(agg, deg3, self_msg, b, lin_w, lin_b):
    grid = (N_NODES // ROW_BLK,)
    return pl.pallas_call(
        _final_body,
        grid=grid,
        in_specs=[
            pl.BlockSpec((ROW_BLK, D_H), lambda i: (i, 0)),
            pl.BlockSpec((ROW_BLK, 1), lambda i: (i, 0)),
            pl.BlockSpec((ROW_BLK, D_H), lambda i: (i, 0)),
            pl.BlockSpec((D_H,), lambda i: (0,)),
            pl.BlockSpec((D_H, N_OUT), lambda i: (0, 0)),
            pl.BlockSpec((N_OUT,), lambda i: (0,)),
        ],
        out_specs=pl.BlockSpec((ROW_BLK, N_OUT), lambda i: (i, 0)),
        out_shape=jax.ShapeDtypeStruct((N_NODES, N_OUT), jnp.float32),
    )(agg, deg3, self_msg, b, lin_w, lin_b)


def _edge_phase(x, U, c, W, edges):
    src, dst = edges[0], edges[1]
    mask = (src != dst).astype(jnp.float32)
    xb = x.astype(jnp.bfloat16)
    xj = xb[src]
    xi = xb[dst]
    logits = jnp.matmul(xj - xi, U.astype(jnp.bfloat16),
                        preferred_element_type=jnp.float32) + c
    q = jax.nn.softmax(logits, axis=1)
    q = q * mask[:, None]
    m = jnp.matmul(xj, W.astype(jnp.bfloat16),
                   preferred_element_type=jnp.float32)
    m = m.reshape(-1, HEADS, D_H)
    msg = (m * q[:, :, None]).sum(axis=1)
    agg = jax.ops.segment_sum(msg, dst, num_segments=N_NODES)
    deg = jax.ops.segment_sum(mask, dst, num_segments=N_NODES)
    return agg, deg[:, None]


def _prep_layer(W, U, c):
    Upad = jnp.concatenate([U, U], axis=1)
    cvec = jnp.concatenate([jnp.zeros((HEADS,), jnp.float32), c])
    qs = jax.nn.softmax(c)
    Wself = (W.reshape(D_H, HEADS, D_H) * qs[None, :, None]).sum(axis=1)
    return Upad, cvec, Wself


def kernel(verts, edges, fc0_w, fc0_b, conv1_W, conv1_U, conv1_c, conv1_b,
           conv2_W, conv2_U, conv2_c, conv2_b, lin_w, lin_b):
    Upad1, cvec1, Wself1 = _prep_layer(conv1_W, conv1_U, conv1_c)
    Upad2, cvec2, Wself2 = _prep_layer(conv2_W, conv2_U, conv2_c)

    x0, pq1, self1 = _prologue1(verts, fc0_w, fc0_b, Upad1, cvec1, Wself1)
    agg1, deg1 = _edge_phase(x0, conv1_U, conv1_c, conv1_W, edges)
    x1, pq2, self2 = _mid(agg1, deg1, self1, conv1_b, Upad2, cvec2, Wself2)
    agg2, deg2 = _edge_phase(x1, conv2_U, conv2_c, conv2_W, edges)
    return _final(agg2, deg2, self2, conv2_b, lin_w, lin_b)
